# baseline (speedup 1.0000x reference)
import functools

import jax
import jax.numpy as jnp
from jax import lax
from jax.experimental import pallas as pl
from jax.experimental.pallas import tpu as pltpu
from jax.experimental.pallas import tpu_sc as plsc

_L = 16
_W = 512


def _sc_lookup(xt_flat, tab_flat, s, b, d):
    nc, ns = 2, 16
    hk = d // 2
    mesh = plsc.VectorSubcoreMesh(core_axis_name="c", subcore_axis_name="s")

    @functools.partial(
        pl.kernel,
        mesh=mesh,
        out_type=jax.ShapeDtypeStruct((s, d, b), jnp.float32),
        scratch_types=[
            pltpu.VMEM((tab_flat.shape[0],), jnp.float32),
            pltpu.VMEM((_W,), jnp.int32),
            pltpu.VMEM((d, _W), jnp.float32),
            [pltpu.SemaphoreType.DMA] * 2,
            pltpu.SemaphoreType.DMA,
        ],
        compiler_params=pltpu.CompilerParams(needs_layout_passes=False),
    )
    def k(xt_hbm, tab_hbm, out_hbm, tab_v, idx_v, ov, wsems, isem):
        wid = lax.axis_index("s") * nc + lax.axis_index("c")
        lane0 = wid * _W
        pltpu.sync_copy(tab_hbm, tab_v)

        def out_fire(j, half):
            pltpu.async_copy(
                ov.at[pl.ds(half * hk, hk)],
                out_hbm.at[j, pl.ds(half * hk, hk), pl.ds(lane0, _W)],
                wsems[half],
            )

        def out_wait(j, half):
            pltpu.make_async_copy(
                ov.at[pl.ds(half * hk, hk)],
                out_hbm.at[j, pl.ds(half * hk, hk), pl.ds(lane0, _W)],
                wsems[half],
            ).wait()

        def compute(half):
            for g in range(_W // _L):
                col = g * _L
                base = idx_v[pl.ds(col, _L)] * d
                for kk in range(half * hk, half * hk + hk):
                    ov[kk, pl.ds(col, _L)] = plsc.load_gather(
                        tab_v, [base + kk]
                    )

        def body(j, carry):
            pltpu.sync_copy(
                xt_hbm.at[pl.ds(j * b + lane0, _W)], idx_v
            )
            lax.cond(j > 0, lambda: out_wait(j - 1, 0), lambda: None)
            compute(0)
            out_fire(j, 0)
            lax.cond(j > 0, lambda: out_wait(j - 1, 1), lambda: None)
            compute(1)
            out_fire(j, 1)
            return carry

        lax.fori_loop(0, s, body, 0)
        out_wait(s - 1, 0)
        out_wait(s - 1, 1)

    return k(xt_flat, tab_flat)


def kernel(x, HOURE):
    b0, b1 = x.shape
    v, d = HOURE.shape
    xt_flat = x.T.reshape(b0 * b1)
    tab_flat = HOURE.reshape(v * d)
    out_t = _sc_lookup(xt_flat, tab_flat, b1, b0, d)
    return jnp.transpose(out_t, (2, 0, 1))


# speedup vs baseline: 1.6972x; 1.6972x over previous
import functools

import jax
import jax.numpy as jnp
from jax import lax
from jax.experimental import pallas as pl
from jax.experimental.pallas import tpu as pltpu
from jax.experimental.pallas import tpu_sc as plsc

_L = 16
_W = 512


def _sc_lookup(xt_flat, tab_flat, s, b, d):
    nc, ns = 2, 16
    hk = d // 2
    mesh = plsc.VectorSubcoreMesh(core_axis_name="c", subcore_axis_name="s")

    @functools.partial(
        pl.kernel,
        mesh=mesh,
        out_type=jax.ShapeDtypeStruct((s, d, b), jnp.float32),
        scratch_types=[
            pltpu.VMEM((tab_flat.shape[0],), jnp.float32),
            pltpu.VMEM((_W,), jnp.int32),
            pltpu.VMEM((d, _W), jnp.float32),
            [pltpu.SemaphoreType.DMA] * 2,
            pltpu.SemaphoreType.DMA,
        ],
        compiler_params=pltpu.CompilerParams(needs_layout_passes=False),
    )
    def k(xt_hbm, tab_hbm, out_hbm, tab_v, idx_v, ov, wsems, isem):
        wid = lax.axis_index("s") * nc + lax.axis_index("c")
        lane0 = wid * _W
        pltpu.sync_copy(tab_hbm, tab_v)

        def out_fire(j, half):
            pltpu.async_copy(
                ov.at[pl.ds(half * hk, hk)],
                out_hbm.at[j, pl.ds(half * hk, hk), pl.ds(lane0, _W)],
                wsems[half],
            )

        def out_wait(j, half):
            pltpu.make_async_copy(
                ov.at[pl.ds(half * hk, hk)],
                out_hbm.at[j, pl.ds(half * hk, hk), pl.ds(lane0, _W)],
                wsems[half],
            ).wait()

        def compute(half):
            for g in range(_W // _L):
                col = g * _L
                base = idx_v[pl.ds(col, _L)] * d
                for k0 in range(half * hk, half * hk + hk, 8):
                    vals = [
                        plsc.load_gather(tab_v, [base + (k0 + u)])
                        for u in range(8)
                    ]
                    for u in range(8):
                        ov[k0 + u, pl.ds(col, _L)] = vals[u]

        def body(j, carry):
            pltpu.sync_copy(
                xt_hbm.at[pl.ds(j * b + lane0, _W)], idx_v
            )
            lax.cond(j > 0, lambda: out_wait(j - 1, 0), lambda: None)
            compute(0)
            out_fire(j, 0)
            lax.cond(j > 0, lambda: out_wait(j - 1, 1), lambda: None)
            compute(1)
            out_fire(j, 1)
            return carry

        lax.fori_loop(0, s, body, 0)
        out_wait(s - 1, 0)
        out_wait(s - 1, 1)

    return k(xt_flat, tab_flat)


def kernel(x, HOURE):
    b0, b1 = x.shape
    v, d = HOURE.shape
    xt_flat = x.T.reshape(b0 * b1)
    tab_flat = HOURE.reshape(v * d)
    out_t = _sc_lookup(xt_flat, tab_flat, b1, b0, d)
    return jnp.transpose(out_t, (2, 0, 1))
